# Initial kernel scaffold; baseline (speedup 1.0000x reference)
import jax
import jax.numpy as jnp
from jax.experimental import pallas as pl


def kernel(params, word_norm, topic_norm, w_wt, w_wd, w_td, y_data, word_id, topic_id, edge_ww, edge_tt, wt_src, wt_dst, wd_src, wd_dst, td_src, td_dst, doc_graph_id):
    raise NotImplementedError("write your pallas kernel here")


# speedup vs baseline: 2.7213x; 2.7213x over previous
import functools

import jax
import jax.numpy as jnp
from jax import lax
from jax.experimental import pallas as pl
from jax.experimental.pallas import tpu as pltpu
from jax.experimental.pallas import tpu_sc as plsc

N_W = 10000
N_T = 800
N_D = 3200
B = 16
D = 128

NC = 2
NS = 16
NWORK = NC * NS
LANES = 16

PW = 10240
PT = 1024
PD = 3328

DC = 2 * D


def _mesh():
    return plsc.VectorSubcoreMesh(
        core_axis_name="c", subcore_axis_name="s",
        num_cores=NC, num_subcores=NS)


def _worker_id():
    cid = lax.axis_index("c")
    sid = lax.axis_index("s")
    return cid, sid


def _sc_gather(table, idx, n_pad, chunk):
    rows_per_w = n_pad // NWORK
    nch = rows_per_w // chunk

    @functools.partial(
        pl.kernel,
        out_type=jax.ShapeDtypeStruct((n_pad, D), jnp.float32),
        mesh=_mesh(),
        scratch_types=[
            pltpu.VMEM((chunk,), jnp.int32),
            pltpu.VMEM((chunk, D), jnp.float32),
            pltpu.SemaphoreType.DMA,
        ],
    )
    def k(table_hbm, idx_hbm, out_hbm, idx_v, rows_v, sem):
        cid, sid = _worker_id()
        base = (cid * NS + sid) * rows_per_w

        def body(i, carry):
            off = base + i * chunk
            pltpu.sync_copy(idx_hbm.at[pl.ds(off, chunk)], idx_v)
            pltpu.async_copy(table_hbm.at[idx_v], rows_v, sem).wait()
            pltpu.sync_copy(rows_v, out_hbm.at[pl.ds(off, chunk)])
            return carry

        lax.fori_loop(0, nch, body, 0)

    return k(table, idx)


def _sc_agg_plain(h, src, dst, zeros, n_pad, chunk):
    e = src.shape[0]
    e_per_w = e // NWORK
    nch = e_per_w // chunk
    rows_per_s = n_pad // NS

    @functools.partial(
        pl.kernel,
        out_type=jax.ShapeDtypeStruct((NC, n_pad, D), jnp.float32),
        mesh=_mesh(),
        scratch_types=[
            pltpu.VMEM((chunk,), jnp.int32),
            pltpu.VMEM((chunk,), jnp.int32),
            pltpu.VMEM((chunk, D), jnp.float32),
            pltpu.VMEM_SHARED((n_pad, D), jnp.float32),
            pltpu.SemaphoreType.DMA,
        ],
    )
    def k(h_hbm, src_hbm, dst_hbm, z_hbm, out_hbm,
          idx_s, idx_d, rows_v, accum, sem):
        cid, sid = _worker_id()
        rbase = sid * rows_per_s
        pltpu.sync_copy(z_hbm.at[pl.ds(rbase, rows_per_s)],
                        accum.at[pl.ds(rbase, rows_per_s)])
        plsc.subcore_barrier()

        ebase = (cid * NS + sid) * e_per_w

        def body(i, carry):
            off = ebase + i * chunk
            pltpu.sync_copy(src_hbm.at[pl.ds(off, chunk)], idx_s)
            pltpu.async_copy(h_hbm.at[idx_s], rows_v, sem).wait()
            pltpu.sync_copy(dst_hbm.at[pl.ds(off, chunk)], idx_d)
            pltpu.sync_copy(rows_v, accum.at[idx_d], add=True)
            return carry

        lax.fori_loop(0, nch, body, 0)
        plsc.subcore_barrier()
        pltpu.sync_copy(accum.at[pl.ds(rbase, rows_per_s)],
                        out_hbm.at[cid, pl.ds(rbase, rows_per_s)])

    return k(h, src, dst, zeros)


def _sc_agg_weighted(h, src, dst, w, zeros, msg_init, n_pad, chunk):
    e = src.shape[0]
    e_per_w = e // NWORK
    nch = e_per_w // chunk
    rows_per_s = n_pad // NS

    @functools.partial(
        pl.kernel,
        out_type=jax.ShapeDtypeStruct((NC, n_pad, 2, D), jnp.float32),
        mesh=_mesh(),
        scratch_types=[
            pltpu.VMEM((chunk,), jnp.int32),
            pltpu.VMEM((chunk,), jnp.int32),
            pltpu.VMEM((chunk, LANES), jnp.float32),
            pltpu.VMEM((chunk, D), jnp.float32),
            pltpu.VMEM((chunk, 2, D), jnp.float32),
            pltpu.VMEM_SHARED((n_pad, 2, D), jnp.float32),
            pltpu.SemaphoreType.DMA,
        ],
    )
    def k(h_hbm, src_hbm, dst_hbm, w_hbm, z_hbm, mi_hbm, out_hbm,
          idx_s, idx_d, w_v, rows_v, msg_v, accum, sem):
        cid, sid = _worker_id()
        rbase = sid * rows_per_s
        pltpu.sync_copy(z_hbm.at[pl.ds(rbase, rows_per_s)],
                        accum.at[pl.ds(rbase, rows_per_s)])

        pltpu.sync_copy(mi_hbm, msg_v)
        plsc.subcore_barrier()

        ebase = (cid * NS + sid) * e_per_w

        def body(i, carry):
            off = ebase + i * chunk
            pltpu.sync_copy(src_hbm.at[pl.ds(off, chunk)], idx_s)
            pltpu.sync_copy(w_hbm.at[pl.ds(off, chunk)], w_v)
            pltpu.async_copy(h_hbm.at[idx_s], rows_v, sem).wait()
            for r in range(chunk):
                wrow = w_v[r, pl.ds(0, LANES)]
                for dblk in range(D // LANES):
                    sl = pl.ds(dblk * LANES, LANES)
                    msg_v[r, 0, sl] = rows_v[r, sl] * wrow
            pltpu.sync_copy(dst_hbm.at[pl.ds(off, chunk)], idx_d)
            pltpu.sync_copy(msg_v, accum.at[idx_d], add=True)
            return carry

        lax.fori_loop(0, nch, body, 0)
        plsc.subcore_barrier()
        pltpu.sync_copy(accum.at[pl.ds(rbase, rows_per_s)],
                        out_hbm.at[cid, pl.ds(rbase, rows_per_s)])

    return k(h, src, dst, w, zeros, msg_init)


_BLK = 512


def _tc_mm_scale(x, wmat, scale):
    n = x.shape[0]
    dout = wmat.shape[1]

    def body(x_ref, w_ref, s_ref, o_ref):
        o_ref[...] = jnp.dot(x_ref[...], w_ref[...],
                             preferred_element_type=jnp.float32) * s_ref[...]

    return pl.pallas_call(
        body,
        grid=(n // _BLK,),
        in_specs=[
            pl.BlockSpec((_BLK, x.shape[1]), lambda i: (i, 0)),
            pl.BlockSpec(wmat.shape, lambda i: (0, 0)),
            pl.BlockSpec((_BLK, 1), lambda i: (i, 0)),
        ],
        out_specs=pl.BlockSpec((_BLK, dout), lambda i: (i, 0)),
        out_shape=jax.ShapeDtypeStruct((n, dout), jnp.float32),
    )(x, wmat, scale)


def _tc_post_mm(parts, scale, bias, wmat, out_scale, out_bias):
    n = parts.shape[1]
    dout = wmat.shape[1]

    def body(p_ref, s_ref, b_ref, w_ref, *rest):
        o_ref = rest[-1]
        h = jax.nn.relu((p_ref[0] + p_ref[1]) * s_ref[...] + b_ref[...])
        o = jnp.dot(h, w_ref[...], preferred_element_type=jnp.float32)
        j = 0
        if out_scale is not None:
            o = o * rest[j][...]
            j += 1
        if out_bias is not None:
            o = o + rest[j][...]
        o_ref[...] = o

    in_specs = [
        pl.BlockSpec((NC, _BLK, D), lambda i: (0, i, 0)),
        pl.BlockSpec((_BLK, 1), lambda i: (i, 0)),
        pl.BlockSpec((1, D), lambda i: (0, 0)),
        pl.BlockSpec(wmat.shape, lambda i: (0, 0)),
    ]
    args = [parts, scale, bias, wmat]
    if out_scale is not None:
        in_specs.append(pl.BlockSpec((_BLK, 1), lambda i: (i, 0)))
        args.append(out_scale)
    if out_bias is not None:
        in_specs.append(pl.BlockSpec((1, dout), lambda i: (0, 0)))
        args.append(out_bias)

    return pl.pallas_call(
        body,
        grid=(n // _BLK,),
        in_specs=in_specs,
        out_specs=pl.BlockSpec((_BLK, dout), lambda i: (i, 0)),
        out_shape=jax.ShapeDtypeStruct((n, dout), jnp.float32),
    )(*args)


def _tc_mean_mm(parts, wmat, bias):
    n = parts.shape[1]
    dout = wmat.shape[1]

    def body(p_ref, w_ref, b_ref, o_ref):
        s = p_ref[0] + p_ref[1]
        cnt = jnp.maximum(s[:, D:D + 1], 1.0)
        mean = s[:, :D] / cnt
        o_ref[...] = jnp.dot(mean, w_ref[...],
                             preferred_element_type=jnp.float32) + b_ref[...]

    return pl.pallas_call(
        body,
        grid=(),
        in_specs=[
            pl.BlockSpec(parts.shape, lambda: (0, 0, 0)),
            pl.BlockSpec(wmat.shape, lambda: (0, 0)),
            pl.BlockSpec((1, dout), lambda: (0, 0)),
        ],
        out_specs=pl.BlockSpec((n, dout), lambda: (0, 0)),
        out_shape=jax.ShapeDtypeStruct((n, dout), jnp.float32),
    )(parts, wmat, bias)


def _tc_final(pwd, ptd, ids, out_w, out_b, y):
    def body(pwd_ref, ptd_ref, ids_ref, w_ref, b_ref, y_ref,
             loss_ref, pred_ref):
        swd = pwd_ref[0] + pwd_ref[1]
        std = ptd_ref[0] + ptd_ref[1]
        doc = jax.nn.relu(
            swd[:, :D] / jnp.maximum(swd[:, D:D + 1], 1.0)
            + std[:, :D] / jnp.maximum(std[:, D:D + 1], 1.0))
        ids = ids_ref[...]
        neg = jnp.float32(-jnp.inf)
        pooled = jnp.stack(
            [jnp.max(jnp.where(ids == b, doc, neg), axis=0)
             for b in range(B)], axis=0)
        z = jnp.sum(pooled * w_ref[...].reshape(1, D), axis=1,
                    keepdims=True) + b_ref[...]
        yv = y_ref[...]
        loss = jnp.mean(jnp.maximum(z, 0.0) - z * yv
                        + jnp.log(1.0 + jnp.exp(-jnp.abs(z))))
        loss_ref[...] = jnp.reshape(loss, (1, 1))
        pred_ref[...] = 1.0 / (1.0 + jnp.exp(-z))

    return pl.pallas_call(
        body,
        grid=(),
        in_specs=[
            pl.BlockSpec(pwd.shape, lambda: (0, 0, 0)),
            pl.BlockSpec(ptd.shape, lambda: (0, 0, 0)),
            pl.BlockSpec(ids.shape, lambda: (0, 0)),
            pl.BlockSpec((D, 1), lambda: (0, 0)),
            pl.BlockSpec((1, 1), lambda: (0, 0)),
            pl.BlockSpec((B, 1), lambda: (0, 0)),
        ],
        out_specs=[
            pl.BlockSpec((1, 1), lambda: (0, 0)),
            pl.BlockSpec((B, 1), lambda: (0, 0)),
        ],
        out_shape=[
            jax.ShapeDtypeStruct((1, 1), jnp.float32),
            jax.ShapeDtypeStruct((B, 1), jnp.float32),
        ],
    )(pwd, ptd, ids, out_w, out_b, y)


def kernel(params, word_norm, topic_norm, w_wt, w_wd, w_td, y_data,
           word_id, topic_id, edge_ww, edge_tt, wt_src, wt_dst,
           wd_src, wd_dst, td_src, td_dst, doc_graph_id):
    f32 = jnp.float32
    i32 = jnp.int32

    word_id_p = jnp.concatenate(
        [word_id.astype(i32), jnp.zeros((PW - N_W,), i32)])
    norm_p = jnp.concatenate(
        [word_norm, jnp.zeros((PW - N_W,), f32)]).reshape(PW, 1)
    ids_p = jnp.concatenate(
        [doc_graph_id.astype(i32),
         jnp.full((PD - N_D,), -1, i32)]).reshape(PD, 1)

    z_w = jnp.zeros((PW, D), f32)
    z_t = jnp.zeros((PT, 2, D), f32)
    z_d = jnp.zeros((PD, 2, D), f32)
    msg_init = jnp.zeros((40, 2, D), f32).at[:, 1, 0].set(1.0)

    word_h0 = _sc_gather(params['word_embeds'], word_id_p, PW, chunk=64)

    t0 = _tc_mm_scale(word_h0, params['gw_W0'], norm_p)
    p0 = _sc_agg_plain(t0, edge_ww[0].astype(i32), edge_ww[1].astype(i32),
                       z_w, PW, chunk=80)

    t1 = _tc_post_mm(p0, norm_p, params['gw_b0'].reshape(1, D),
                     params['gw_W1'], out_scale=norm_p, out_bias=None)
    p1 = _sc_agg_plain(t1, edge_ww[0].astype(i32), edge_ww[1].astype(i32),
                       z_w, PW, chunk=80)

    wcat = jnp.concatenate([params['h0_wt_W'], params['h1_wd_W']], axis=1)
    bcat = jnp.concatenate([params['h0_wt_b'], params['h1_wd_b']]).reshape(1, 2 * D)
    wh = _tc_post_mm(p1, norm_p, params['gw_b1'].reshape(1, D),
                     wcat, out_scale=None, out_bias=bcat)
    wh_wt = wh[:, :D]
    wh_wd = wh[:, D:]

    w_wt_b = jnp.broadcast_to(w_wt[:, None], (w_wt.shape[0], LANES))
    w_wd_b = jnp.broadcast_to(w_wd[:, None], (w_wd.shape[0], LANES))
    w_td_b = jnp.broadcast_to(w_td[:, None], (w_td.shape[0], LANES))

    pt = _sc_agg_weighted(wh_wt, wt_src.astype(i32), wt_dst.astype(i32),
                          w_wt_b, z_t, msg_init, PT, chunk=40)
    wh_td = _tc_mean_mm(pt.reshape(NC, PT, DC), params['h1_td_W'],
                        params['h1_td_b'].reshape(1, D))

    pwd = _sc_agg_weighted(wh_wd, wd_src.astype(i32), wd_dst.astype(i32),
                           w_wd_b, z_d, msg_init, PD, chunk=40)
    ptd = _sc_agg_weighted(wh_td, td_src.astype(i32), td_dst.astype(i32),
                           w_td_b, z_d, msg_init, PD, chunk=40)

    loss, y_pred = _tc_final(pwd.reshape(NC, PD, DC), ptd.reshape(NC, PD, DC),
                             ids_p,
                             params['out_W'].reshape(D, 1),
                             params['out_b'].reshape(1, 1),
                             y_data.reshape(B, 1))
    return loss.reshape(()), y_pred


# speedup vs baseline: 3.4636x; 1.2728x over previous
import functools

import jax
import jax.numpy as jnp
from jax import lax
from jax.experimental import pallas as pl
from jax.experimental.pallas import tpu as pltpu
from jax.experimental.pallas import tpu_sc as plsc

N_W = 10000
N_T = 800
N_D = 3200
B = 16
D = 128

NC = 2
NS = 16
NWORK = NC * NS
LANES = 16

PW = 10240
PT = 1024
PD = 3328

DC = 2 * D


def _mesh():
    return plsc.VectorSubcoreMesh(
        core_axis_name="c", subcore_axis_name="s",
        num_cores=NC, num_subcores=NS)


def _worker_id():
    cid = lax.axis_index("c")
    sid = lax.axis_index("s")
    return cid, sid


def _sc_gather(table, idx, n_pad, chunk):
    rows_per_w = n_pad // NWORK
    nch = rows_per_w // chunk

    @functools.partial(
        pl.kernel,
        out_type=jax.ShapeDtypeStruct((n_pad, D), jnp.float32),
        mesh=_mesh(),
        scratch_types=[
            pltpu.VMEM((chunk,), jnp.int32),
            pltpu.VMEM((chunk, D), jnp.float32),
            pltpu.SemaphoreType.DMA,
        ],
    )
    def k(table_hbm, idx_hbm, out_hbm, idx_v, rows_v, sem):
        cid, sid = _worker_id()
        base = (cid * NS + sid) * rows_per_w

        def body(i, carry):
            off = base + i * chunk
            pltpu.sync_copy(idx_hbm.at[pl.ds(off, chunk)], idx_v)
            pltpu.async_copy(table_hbm.at[idx_v], rows_v, sem).wait()
            pltpu.sync_copy(rows_v, out_hbm.at[pl.ds(off, chunk)])
            return carry

        lax.fori_loop(0, nch, body, 0)

    return k(table, idx)


def _sc_agg_plain(h, src, dst, zeros, n_pad, chunk):
    e = src.shape[0]
    e_per_w = e // NWORK
    nch = e_per_w // chunk
    nsup = nch // 3
    rows_per_s = n_pad // NS

    @functools.partial(
        pl.kernel,
        out_type=jax.ShapeDtypeStruct((NC, n_pad, D), jnp.float32),
        mesh=_mesh(),
        scratch_types=(
            [pltpu.VMEM((chunk,), jnp.int32)] * 3
            + [pltpu.VMEM((chunk,), jnp.int32)] * 3
            + [pltpu.VMEM((chunk, D), jnp.float32)] * 3
            + [pltpu.VMEM_SHARED((n_pad, D), jnp.float32)]
            + [pltpu.SemaphoreType.DMA] * 6
        ),
    )
    def k(h_hbm, src_hbm, dst_hbm, z_hbm, out_hbm, *scr):
        idx_s = scr[0:3]
        idx_d = scr[3:6]
        rows = scr[6:9]
        accum = scr[9]
        sem_g = scr[10:13]
        sem_s = scr[13:16]

        cid, sid = _worker_id()
        rbase = sid * rows_per_s
        pltpu.sync_copy(z_hbm.at[pl.ds(rbase, rows_per_s)],
                        accum.at[pl.ds(rbase, rows_per_s)])
        plsc.subcore_barrier()

        ebase = (cid * NS + sid) * e_per_w

        def load_and_gather(i, b):
            off = ebase + i * chunk
            pltpu.sync_copy(src_hbm.at[pl.ds(off, chunk)], idx_s[b])
            pltpu.sync_copy(dst_hbm.at[pl.ds(off, chunk)], idx_d[b])
            pltpu.async_copy(h_hbm.at[idx_s[b]], rows[b], sem_g[b])

        def wait_gather(b):
            pltpu.make_async_copy(h_hbm.at[idx_s[b]], rows[b],
                                  sem_g[b]).wait()

        def wait_scatter(b):
            pltpu.make_async_copy(rows[b], accum.at[idx_d[b]],
                                  sem_s[b]).wait()

        load_and_gather(0, 0)
        load_and_gather(1, 1)

        def sup(j, carry):
            for kk in range(3):
                i = 3 * j + kk
                b = kk
                bp = (kk + 2) % 3
                if kk == 0:
                    @pl.when(j > 0)
                    def _():
                        wait_scatter(bp)
                else:
                    wait_scatter(bp)
                if kk == 0:
                    load_and_gather(i + 2, bp)
                else:
                    @pl.when(j < nsup - 1)
                    def _():
                        load_and_gather(i + 2, bp)
                wait_gather(b)
                pltpu.async_copy(rows[b], accum.at[idx_d[b]], sem_s[b],
                                 add=True)
            return carry

        lax.fori_loop(0, nsup, sup, 0)
        wait_scatter((nch - 1) % 3)
        plsc.subcore_barrier()
        pltpu.sync_copy(accum.at[pl.ds(rbase, rows_per_s)],
                        out_hbm.at[cid, pl.ds(rbase, rows_per_s)])

    return k(h, src, dst, zeros)


def _sc_agg_weighted(h, src, dst, w, zeros, msg_init, n_pad, chunk):
    e = src.shape[0]
    e_per_w = e // NWORK
    nch = e_per_w // chunk
    rows_per_s = n_pad // NS

    nsup = nch // 3

    @functools.partial(
        pl.kernel,
        out_type=jax.ShapeDtypeStruct((NC, n_pad, 2, D), jnp.float32),
        mesh=_mesh(),
        scratch_types=(
            [pltpu.VMEM((chunk,), jnp.int32)] * 3
            + [pltpu.VMEM((chunk,), jnp.int32)] * 3
            + [pltpu.VMEM((chunk, LANES), jnp.float32)] * 3
            + [pltpu.VMEM((chunk, D), jnp.float32)] * 3
            + [pltpu.VMEM((chunk, 2, D), jnp.float32)] * 3
            + [pltpu.VMEM_SHARED((n_pad, 2, D), jnp.float32)]
            + [pltpu.SemaphoreType.DMA] * 6
        ),
    )
    def k(h_hbm, src_hbm, dst_hbm, w_hbm, z_hbm, mi_hbm, out_hbm, *scr):
        idx_s = scr[0:3]
        idx_d = scr[3:6]
        w_v = scr[6:9]
        rows = scr[9:12]
        msg = scr[12:15]
        accum = scr[15]
        sem_g = scr[16:19]
        sem_s = scr[19:22]

        cid, sid = _worker_id()
        rbase = sid * rows_per_s
        pltpu.sync_copy(z_hbm.at[pl.ds(rbase, rows_per_s)],
                        accum.at[pl.ds(rbase, rows_per_s)])

        for b in range(3):
            pltpu.sync_copy(mi_hbm, msg[b])
        plsc.subcore_barrier()

        ebase = (cid * NS + sid) * e_per_w

        def load_and_gather(i, b):
            off = ebase + i * chunk
            pltpu.sync_copy(src_hbm.at[pl.ds(off, chunk)], idx_s[b])
            pltpu.sync_copy(dst_hbm.at[pl.ds(off, chunk)], idx_d[b])
            pltpu.sync_copy(w_hbm.at[pl.ds(off, chunk)], w_v[b])
            pltpu.async_copy(h_hbm.at[idx_s[b]], rows[b], sem_g[b])

        def wait_gather(b):
            pltpu.make_async_copy(h_hbm.at[idx_s[b]], rows[b],
                                  sem_g[b]).wait()

        def wait_scatter(b):
            pltpu.make_async_copy(msg[b], accum.at[idx_d[b]],
                                  sem_s[b]).wait()

        load_and_gather(0, 0)
        load_and_gather(1, 1)

        def sup(j, carry):
            for kk in range(3):
                i = 3 * j + kk
                b = kk
                bp = (kk + 2) % 3
                if kk == 0:
                    @pl.when(j > 0)
                    def _():
                        wait_scatter(bp)
                else:
                    wait_scatter(bp)
                if kk == 0:
                    load_and_gather(i + 2, bp)
                else:
                    @pl.when(j < nsup - 1)
                    def _():
                        load_and_gather(i + 2, bp)
                wait_gather(b)
                for r in range(chunk):
                    wrow = w_v[b][r, pl.ds(0, LANES)]
                    for dblk in range(D // LANES):
                        sl = pl.ds(dblk * LANES, LANES)
                        msg[b][r, 0, sl] = rows[b][r, sl] * wrow
                pltpu.async_copy(msg[b], accum.at[idx_d[b]], sem_s[b],
                                 add=True)
            return carry

        lax.fori_loop(0, nsup, sup, 0)
        wait_scatter((nch - 1) % 3)
        plsc.subcore_barrier()
        pltpu.sync_copy(accum.at[pl.ds(rbase, rows_per_s)],
                        out_hbm.at[cid, pl.ds(rbase, rows_per_s)])

    return k(h, src, dst, w, zeros, msg_init)


_BLK = 512


def _tc_mm_scale(x, wmat, scale):
    n = x.shape[0]
    dout = wmat.shape[1]

    def body(x_ref, w_ref, s_ref, o_ref):
        o_ref[...] = jnp.dot(x_ref[...], w_ref[...],
                             preferred_element_type=jnp.float32) * s_ref[...]

    return pl.pallas_call(
        body,
        grid=(n // _BLK,),
        in_specs=[
            pl.BlockSpec((_BLK, x.shape[1]), lambda i: (i, 0)),
            pl.BlockSpec(wmat.shape, lambda i: (0, 0)),
            pl.BlockSpec((_BLK, 1), lambda i: (i, 0)),
        ],
        out_specs=pl.BlockSpec((_BLK, dout), lambda i: (i, 0)),
        out_shape=jax.ShapeDtypeStruct((n, dout), jnp.float32),
    )(x, wmat, scale)


def _tc_post_mm(parts, scale, bias, wmat, out_scale, out_bias):
    n = parts.shape[1]
    dout = wmat.shape[1]

    def body(p_ref, s_ref, b_ref, w_ref, *rest):
        o_ref = rest[-1]
        h = jax.nn.relu((p_ref[0] + p_ref[1]) * s_ref[...] + b_ref[...])
        o = jnp.dot(h, w_ref[...], preferred_element_type=jnp.float32)
        j = 0
        if out_scale is not None:
            o = o * rest[j][...]
            j += 1
        if out_bias is not None:
            o = o + rest[j][...]
        o_ref[...] = o

    in_specs = [
        pl.BlockSpec((NC, _BLK, D), lambda i: (0, i, 0)),
        pl.BlockSpec((_BLK, 1), lambda i: (i, 0)),
        pl.BlockSpec((1, D), lambda i: (0, 0)),
        pl.BlockSpec(wmat.shape, lambda i: (0, 0)),
    ]
    args = [parts, scale, bias, wmat]
    if out_scale is not None:
        in_specs.append(pl.BlockSpec((_BLK, 1), lambda i: (i, 0)))
        args.append(out_scale)
    if out_bias is not None:
        in_specs.append(pl.BlockSpec((1, dout), lambda i: (0, 0)))
        args.append(out_bias)

    return pl.pallas_call(
        body,
        grid=(n // _BLK,),
        in_specs=in_specs,
        out_specs=pl.BlockSpec((_BLK, dout), lambda i: (i, 0)),
        out_shape=jax.ShapeDtypeStruct((n, dout), jnp.float32),
    )(*args)


def _tc_mean_mm(parts, wmat, bias):
    n = parts.shape[1]
    dout = wmat.shape[1]

    def body(p_ref, w_ref, b_ref, o_ref):
        s = p_ref[0] + p_ref[1]
        cnt = jnp.maximum(s[:, D:D + 1], 1.0)
        mean = s[:, :D] / cnt
        o_ref[...] = jnp.dot(mean, w_ref[...],
                             preferred_element_type=jnp.float32) + b_ref[...]

    return pl.pallas_call(
        body,
        grid=(),
        in_specs=[
            pl.BlockSpec(parts.shape, lambda: (0, 0, 0)),
            pl.BlockSpec(wmat.shape, lambda: (0, 0)),
            pl.BlockSpec((1, dout), lambda: (0, 0)),
        ],
        out_specs=pl.BlockSpec((n, dout), lambda: (0, 0)),
        out_shape=jax.ShapeDtypeStruct((n, dout), jnp.float32),
    )(parts, wmat, bias)


def _tc_final(pwd, ptd, ids, out_w, out_b, y):
    def body(pwd_ref, ptd_ref, ids_ref, w_ref, b_ref, y_ref,
             loss_ref, pred_ref):
        swd = pwd_ref[0] + pwd_ref[1]
        std = ptd_ref[0] + ptd_ref[1]
        doc = jax.nn.relu(
            swd[:, :D] / jnp.maximum(swd[:, D:D + 1], 1.0)
            + std[:, :D] / jnp.maximum(std[:, D:D + 1], 1.0))
        ids = ids_ref[...]
        neg = jnp.float32(-jnp.inf)
        pooled = jnp.stack(
            [jnp.max(jnp.where(ids == b, doc, neg), axis=0)
             for b in range(B)], axis=0)
        z = jnp.sum(pooled * w_ref[...].reshape(1, D), axis=1,
                    keepdims=True) + b_ref[...]
        yv = y_ref[...]
        loss = jnp.mean(jnp.maximum(z, 0.0) - z * yv
                        + jnp.log(1.0 + jnp.exp(-jnp.abs(z))))
        loss_ref[...] = jnp.reshape(loss, (1, 1))
        pred_ref[...] = 1.0 / (1.0 + jnp.exp(-z))

    return pl.pallas_call(
        body,
        grid=(),
        in_specs=[
            pl.BlockSpec(pwd.shape, lambda: (0, 0, 0)),
            pl.BlockSpec(ptd.shape, lambda: (0, 0, 0)),
            pl.BlockSpec(ids.shape, lambda: (0, 0)),
            pl.BlockSpec((D, 1), lambda: (0, 0)),
            pl.BlockSpec((1, 1), lambda: (0, 0)),
            pl.BlockSpec((B, 1), lambda: (0, 0)),
        ],
        out_specs=[
            pl.BlockSpec((1, 1), lambda: (0, 0)),
            pl.BlockSpec((B, 1), lambda: (0, 0)),
        ],
        out_shape=[
            jax.ShapeDtypeStruct((1, 1), jnp.float32),
            jax.ShapeDtypeStruct((B, 1), jnp.float32),
        ],
    )(pwd, ptd, ids, out_w, out_b, y)


def kernel(params, word_norm, topic_norm, w_wt, w_wd, w_td, y_data,
           word_id, topic_id, edge_ww, edge_tt, wt_src, wt_dst,
           wd_src, wd_dst, td_src, td_dst, doc_graph_id):
    f32 = jnp.float32
    i32 = jnp.int32

    word_id_p = jnp.concatenate(
        [word_id.astype(i32), jnp.zeros((PW - N_W,), i32)])
    norm_p = jnp.concatenate(
        [word_norm, jnp.zeros((PW - N_W,), f32)]).reshape(PW, 1)
    ids_p = jnp.concatenate(
        [doc_graph_id.astype(i32),
         jnp.full((PD - N_D,), -1, i32)]).reshape(PD, 1)

    z_w = jnp.zeros((PW, D), f32)
    z_t = jnp.zeros((PT, 2, D), f32)
    z_d = jnp.zeros((PD, 2, D), f32)
    msg_init = jnp.zeros((48, 2, D), f32).at[:, 1, 0].set(1.0)

    def pad_edges(s, d, wgt, e_pad, trash):
        ecur = s.shape[0]
        s = jnp.concatenate([s.astype(i32), jnp.zeros((e_pad - ecur,), i32)])
        d = jnp.concatenate([d.astype(i32),
                             jnp.full((e_pad - ecur,), trash, i32)])
        if wgt is not None:
            wgt = jnp.concatenate([wgt, jnp.zeros((e_pad - ecur,), f32)])
            wgt = jnp.broadcast_to(wgt[:, None], (e_pad, LANES))
        return s, d, wgt

    ww_s, ww_d, _ = pad_edges(edge_ww[0], edge_ww[1], None, 322560, N_W)
    wt_s, wt_d, wt_w = pad_edges(wt_src, wt_dst, w_wt, 161280, N_T)
    wd_s, wd_d, wd_w = pad_edges(wd_src, wd_dst, w_wd, 322560, N_D)
    td_s, td_d, td_w = pad_edges(td_src, td_dst, w_td, 64512, N_D)

    word_h0 = _sc_gather(params['word_embeds'], word_id_p, PW, chunk=64)

    t0 = _tc_mm_scale(word_h0, params['gw_W0'], norm_p)
    p0 = _sc_agg_plain(t0, ww_s, ww_d, z_w, PW, chunk=96)

    t1 = _tc_post_mm(p0, norm_p, params['gw_b0'].reshape(1, D),
                     params['gw_W1'], out_scale=norm_p, out_bias=None)
    p1 = _sc_agg_plain(t1, ww_s, ww_d, z_w, PW, chunk=96)

    wcat = jnp.concatenate([params['h0_wt_W'], params['h1_wd_W']], axis=1)
    bcat = jnp.concatenate([params['h0_wt_b'], params['h1_wd_b']]).reshape(1, 2 * D)
    wh = _tc_post_mm(p1, norm_p, params['gw_b1'].reshape(1, D),
                     wcat, out_scale=None, out_bias=bcat)
    wh_wt = wh[:, :D]
    wh_wd = wh[:, D:]

    pt = _sc_agg_weighted(wh_wt, wt_s, wt_d, wt_w, z_t, msg_init,
                          PT, chunk=48)
    wh_td = _tc_mean_mm(pt.reshape(NC, PT, DC), params['h1_td_W'],
                        params['h1_td_b'].reshape(1, D))

    pwd = _sc_agg_weighted(wh_wd, wd_s, wd_d, wd_w, z_d, msg_init,
                           PD, chunk=48)
    ptd = _sc_agg_weighted(wh_td, td_s, td_d, td_w, z_d, msg_init,
                           PD, chunk=48)

    loss, y_pred = _tc_final(pwd.reshape(NC, PD, DC), ptd.reshape(NC, PD, DC),
                             ids_p,
                             params['out_W'].reshape(D, 1),
                             params['out_b'].reshape(1, 1),
                             y_data.reshape(B, 1))
    return loss.reshape(()), y_pred


# speedup vs baseline: 3.4965x; 1.0095x over previous
import functools

import jax
import jax.numpy as jnp
from jax import lax
from jax.experimental import pallas as pl
from jax.experimental.pallas import tpu as pltpu
from jax.experimental.pallas import tpu_sc as plsc

N_W = 10000
N_T = 800
N_D = 3200
B = 16
D = 128

NC = 2
NS = 16
NWORK = NC * NS
LANES = 16

PW = 10240
PT = 1024
PD = 3328

DC = 2 * D


def _mesh():
    return plsc.VectorSubcoreMesh(
        core_axis_name="c", subcore_axis_name="s",
        num_cores=NC, num_subcores=NS)


def _worker_id():
    cid = lax.axis_index("c")
    sid = lax.axis_index("s")
    return cid, sid


def _sc_gather(table, idx, n_pad, chunk):
    rows_per_w = n_pad // NWORK
    nch = rows_per_w // chunk

    @functools.partial(
        pl.kernel,
        out_type=jax.ShapeDtypeStruct((n_pad, D), jnp.float32),
        mesh=_mesh(),
        scratch_types=[
            pltpu.VMEM((chunk,), jnp.int32),
            pltpu.VMEM((chunk, D), jnp.float32),
            pltpu.SemaphoreType.DMA,
        ],
    )
    def k(table_hbm, idx_hbm, out_hbm, idx_v, rows_v, sem):
        cid, sid = _worker_id()
        base = (cid * NS + sid) * rows_per_w

        def body(i, carry):
            off = base + i * chunk
            pltpu.sync_copy(idx_hbm.at[pl.ds(off, chunk)], idx_v)
            pltpu.async_copy(table_hbm.at[idx_v], rows_v, sem).wait()
            pltpu.sync_copy(rows_v, out_hbm.at[pl.ds(off, chunk)])
            return carry

        lax.fori_loop(0, nch, body, 0)

    return k(table, idx)


def _sc_agg_plain(h, src, dst, zeros, n_pad, chunk):
    e = src.shape[0]
    e_per_w = e // NWORK
    nch = e_per_w // chunk
    nsup = nch // 3
    rows_per_s = n_pad // NS

    @functools.partial(
        pl.kernel,
        out_type=jax.ShapeDtypeStruct((NC, n_pad, D), jnp.float32),
        mesh=_mesh(),
        scratch_types=(
            [pltpu.VMEM((chunk,), jnp.int32)] * 3
            + [pltpu.VMEM((chunk,), jnp.int32)] * 3
            + [pltpu.VMEM((chunk, D), jnp.float32)] * 3
            + [pltpu.VMEM_SHARED((n_pad, D), jnp.float32)]
            + [pltpu.SemaphoreType.DMA] * 6
        ),
    )
    def k(h_hbm, src_hbm, dst_hbm, z_hbm, out_hbm, *scr):
        idx_s = scr[0:3]
        idx_d = scr[3:6]
        rows = scr[6:9]
        accum = scr[9]
        sem_g = scr[10:13]
        sem_s = scr[13:16]

        cid, sid = _worker_id()
        rbase = sid * rows_per_s
        pltpu.sync_copy(z_hbm.at[pl.ds(rbase, rows_per_s)],
                        accum.at[pl.ds(rbase, rows_per_s)])
        plsc.subcore_barrier()

        ebase = (cid * NS + sid) * e_per_w

        def load_and_gather(i, b):
            off = ebase + i * chunk
            pltpu.sync_copy(src_hbm.at[pl.ds(off, chunk)], idx_s[b])
            pltpu.sync_copy(dst_hbm.at[pl.ds(off, chunk)], idx_d[b])
            pltpu.async_copy(h_hbm.at[idx_s[b]], rows[b], sem_g[b])

        def wait_gather(b):
            pltpu.make_async_copy(h_hbm.at[idx_s[b]], rows[b],
                                  sem_g[b]).wait()

        def wait_scatter(b):
            pltpu.make_async_copy(rows[b], accum.at[idx_d[b]],
                                  sem_s[b]).wait()

        load_and_gather(0, 0)
        load_and_gather(1, 1)

        def sup(j, carry):
            for kk in range(3):
                i = 3 * j + kk
                b = kk
                bp = (kk + 2) % 3
                if kk == 0:
                    @pl.when(j > 0)
                    def _():
                        wait_scatter(bp)
                else:
                    wait_scatter(bp)
                if kk == 0:
                    load_and_gather(i + 2, bp)
                else:
                    @pl.when(j < nsup - 1)
                    def _():
                        load_and_gather(i + 2, bp)
                wait_gather(b)
                pltpu.async_copy(rows[b], accum.at[idx_d[b]], sem_s[b],
                                 add=True)
            return carry

        lax.fori_loop(0, nsup, sup, 0)
        wait_scatter((nch - 1) % 3)
        plsc.subcore_barrier()
        pltpu.sync_copy(accum.at[pl.ds(rbase, rows_per_s)],
                        out_hbm.at[cid, pl.ds(rbase, rows_per_s)])

    return k(h, src, dst, zeros)


def _sc_agg_weighted(h, src, dst, w, zeros, msg_init, n_pad, chunk):
    e = src.shape[0]
    e_per_w = e // NWORK
    nch = e_per_w // chunk
    rows_per_s = n_pad // NS

    nsup = nch // 3

    @functools.partial(
        pl.kernel,
        out_type=jax.ShapeDtypeStruct((NC, n_pad, 2, D), jnp.float32),
        mesh=_mesh(),
        scratch_types=(
            [pltpu.VMEM((chunk,), jnp.int32)] * 3
            + [pltpu.VMEM((chunk,), jnp.int32)] * 3
            + [pltpu.VMEM((chunk, LANES), jnp.float32)] * 3
            + [pltpu.VMEM((chunk, D), jnp.float32)] * 3
            + [pltpu.VMEM((chunk, 2, D), jnp.float32)] * 3
            + [pltpu.VMEM_SHARED((n_pad, 2, D), jnp.float32)]
            + [pltpu.SemaphoreType.DMA] * 6
        ),
    )
    def k(h_hbm, src_hbm, dst_hbm, w_hbm, z_hbm, mi_hbm, out_hbm, *scr):
        idx_s = scr[0:3]
        idx_d = scr[3:6]
        w_v = scr[6:9]
        rows = scr[9:12]
        msg = scr[12:15]
        accum = scr[15]
        sem_g = scr[16:19]
        sem_s = scr[19:22]

        cid, sid = _worker_id()
        rbase = sid * rows_per_s
        pltpu.sync_copy(z_hbm.at[pl.ds(rbase, rows_per_s)],
                        accum.at[pl.ds(rbase, rows_per_s)])

        for b in range(3):
            pltpu.sync_copy(mi_hbm.at[pl.ds(0, chunk)], msg[b])
        plsc.subcore_barrier()

        ebase = (cid * NS + sid) * e_per_w

        def load_and_gather(i, b):
            off = ebase + i * chunk
            pltpu.sync_copy(src_hbm.at[pl.ds(off, chunk)], idx_s[b])
            pltpu.sync_copy(dst_hbm.at[pl.ds(off, chunk)], idx_d[b])
            pltpu.sync_copy(w_hbm.at[pl.ds(off, chunk)], w_v[b])
            pltpu.async_copy(h_hbm.at[idx_s[b]], rows[b], sem_g[b])

        def wait_gather(b):
            pltpu.make_async_copy(h_hbm.at[idx_s[b]], rows[b],
                                  sem_g[b]).wait()

        def wait_scatter(b):
            pltpu.make_async_copy(msg[b], accum.at[idx_d[b]],
                                  sem_s[b]).wait()

        load_and_gather(0, 0)
        load_and_gather(1, 1)

        def sup(j, carry):
            for kk in range(3):
                i = 3 * j + kk
                b = kk
                bp = (kk + 2) % 3
                if kk == 0:
                    @pl.when(j > 0)
                    def _():
                        wait_scatter(bp)
                else:
                    wait_scatter(bp)
                if kk == 0:
                    load_and_gather(i + 2, bp)
                else:
                    @pl.when(j < nsup - 1)
                    def _():
                        load_and_gather(i + 2, bp)
                wait_gather(b)
                for r in range(chunk):
                    wrow = w_v[b][r, pl.ds(0, LANES)]
                    for dblk in range(D // LANES):
                        sl = pl.ds(dblk * LANES, LANES)
                        msg[b][r, 0, sl] = rows[b][r, sl] * wrow
                pltpu.async_copy(msg[b], accum.at[idx_d[b]], sem_s[b],
                                 add=True)
            return carry

        lax.fori_loop(0, nsup, sup, 0)
        wait_scatter((nch - 1) % 3)
        plsc.subcore_barrier()
        pltpu.sync_copy(accum.at[pl.ds(rbase, rows_per_s)],
                        out_hbm.at[cid, pl.ds(rbase, rows_per_s)])

    return k(h, src, dst, w, zeros, msg_init)


_BLK = 512


def _tc_mm_scale(x, wmat, scale):
    n = x.shape[0]
    dout = wmat.shape[1]

    def body(x_ref, w_ref, s_ref, o_ref):
        o_ref[...] = jnp.dot(x_ref[...], w_ref[...],
                             preferred_element_type=jnp.float32) * s_ref[...]

    return pl.pallas_call(
        body,
        grid=(n // _BLK,),
        in_specs=[
            pl.BlockSpec((_BLK, x.shape[1]), lambda i: (i, 0)),
            pl.BlockSpec(wmat.shape, lambda i: (0, 0)),
            pl.BlockSpec((_BLK, 1), lambda i: (i, 0)),
        ],
        out_specs=pl.BlockSpec((_BLK, dout), lambda i: (i, 0)),
        out_shape=jax.ShapeDtypeStruct((n, dout), jnp.float32),
    )(x, wmat, scale)


def _tc_post_mm(parts, scale, bias, wmat, out_scale, out_bias):
    n = parts.shape[1]
    dout = wmat.shape[1]

    def body(p_ref, s_ref, b_ref, w_ref, *rest):
        o_ref = rest[-1]
        h = jax.nn.relu((p_ref[0] + p_ref[1]) * s_ref[...] + b_ref[...])
        o = jnp.dot(h, w_ref[...], preferred_element_type=jnp.float32)
        j = 0
        if out_scale is not None:
            o = o * rest[j][...]
            j += 1
        if out_bias is not None:
            o = o + rest[j][...]
        o_ref[...] = o

    in_specs = [
        pl.BlockSpec((NC, _BLK, D), lambda i: (0, i, 0)),
        pl.BlockSpec((_BLK, 1), lambda i: (i, 0)),
        pl.BlockSpec((1, D), lambda i: (0, 0)),
        pl.BlockSpec(wmat.shape, lambda i: (0, 0)),
    ]
    args = [parts, scale, bias, wmat]
    if out_scale is not None:
        in_specs.append(pl.BlockSpec((_BLK, 1), lambda i: (i, 0)))
        args.append(out_scale)
    if out_bias is not None:
        in_specs.append(pl.BlockSpec((1, dout), lambda i: (0, 0)))
        args.append(out_bias)

    return pl.pallas_call(
        body,
        grid=(n // _BLK,),
        in_specs=in_specs,
        out_specs=pl.BlockSpec((_BLK, dout), lambda i: (i, 0)),
        out_shape=jax.ShapeDtypeStruct((n, dout), jnp.float32),
    )(*args)


def _tc_mean_mm(parts, wmat, bias):
    n = parts.shape[1]
    dout = wmat.shape[1]

    def body(p_ref, w_ref, b_ref, o_ref):
        s = p_ref[0] + p_ref[1]
        cnt = jnp.maximum(s[:, D:D + 1], 1.0)
        mean = s[:, :D] / cnt
        o_ref[...] = jnp.dot(mean, w_ref[...],
                             preferred_element_type=jnp.float32) + b_ref[...]

    return pl.pallas_call(
        body,
        grid=(),
        in_specs=[
            pl.BlockSpec(parts.shape, lambda: (0, 0, 0)),
            pl.BlockSpec(wmat.shape, lambda: (0, 0)),
            pl.BlockSpec((1, dout), lambda: (0, 0)),
        ],
        out_specs=pl.BlockSpec((n, dout), lambda: (0, 0)),
        out_shape=jax.ShapeDtypeStruct((n, dout), jnp.float32),
    )(parts, wmat, bias)


def _tc_final(pwd, ptd, ids, out_w, out_b, y):
    def body(pwd_ref, ptd_ref, ids_ref, w_ref, b_ref, y_ref,
             loss_ref, pred_ref):
        swd = pwd_ref[0] + pwd_ref[1]
        std = ptd_ref[0] + ptd_ref[1]
        doc = jax.nn.relu(
            swd[:, :D] / jnp.maximum(swd[:, D:D + 1], 1.0)
            + std[:, :D] / jnp.maximum(std[:, D:D + 1], 1.0))
        ids = ids_ref[...]
        neg = jnp.float32(-jnp.inf)
        pooled = jnp.stack(
            [jnp.max(jnp.where(ids == b, doc, neg), axis=0)
             for b in range(B)], axis=0)
        z = jnp.sum(pooled * w_ref[...].reshape(1, D), axis=1,
                    keepdims=True) + b_ref[...]
        yv = y_ref[...]
        loss = jnp.mean(jnp.maximum(z, 0.0) - z * yv
                        + jnp.log(1.0 + jnp.exp(-jnp.abs(z))))
        loss_ref[...] = jnp.reshape(loss, (1, 1))
        pred_ref[...] = 1.0 / (1.0 + jnp.exp(-z))

    return pl.pallas_call(
        body,
        grid=(),
        in_specs=[
            pl.BlockSpec(pwd.shape, lambda: (0, 0, 0)),
            pl.BlockSpec(ptd.shape, lambda: (0, 0, 0)),
            pl.BlockSpec(ids.shape, lambda: (0, 0)),
            pl.BlockSpec((D, 1), lambda: (0, 0)),
            pl.BlockSpec((1, 1), lambda: (0, 0)),
            pl.BlockSpec((B, 1), lambda: (0, 0)),
        ],
        out_specs=[
            pl.BlockSpec((1, 1), lambda: (0, 0)),
            pl.BlockSpec((B, 1), lambda: (0, 0)),
        ],
        out_shape=[
            jax.ShapeDtypeStruct((1, 1), jnp.float32),
            jax.ShapeDtypeStruct((B, 1), jnp.float32),
        ],
    )(pwd, ptd, ids, out_w, out_b, y)


def kernel(params, word_norm, topic_norm, w_wt, w_wd, w_td, y_data,
           word_id, topic_id, edge_ww, edge_tt, wt_src, wt_dst,
           wd_src, wd_dst, td_src, td_dst, doc_graph_id):
    f32 = jnp.float32
    i32 = jnp.int32

    word_id_p = jnp.concatenate(
        [word_id.astype(i32), jnp.zeros((PW - N_W,), i32)])
    norm_p = jnp.concatenate(
        [word_norm, jnp.zeros((PW - N_W,), f32)]).reshape(PW, 1)
    ids_p = jnp.concatenate(
        [doc_graph_id.astype(i32),
         jnp.full((PD - N_D,), -1, i32)]).reshape(PD, 1)

    z_w = jnp.zeros((PW, D), f32)
    z_t = jnp.zeros((PT, 2, D), f32)
    z_d = jnp.zeros((PD, 2, D), f32)
    msg_init = jnp.zeros((80, 2, D), f32).at[:, 1, 0].set(1.0)

    def pad_edges(s, d, wgt, e_pad, trash, ntrash):
        ecur = s.shape[0]
        npad = e_pad - ecur
        s = jnp.concatenate([s.astype(i32), jnp.zeros((npad,), i32)])
        d = jnp.concatenate(
            [d.astype(i32),
             trash + (jnp.arange(npad, dtype=i32) % ntrash)])
        if wgt is not None:
            wgt = jnp.concatenate([wgt, jnp.zeros((npad,), f32)])
            wgt = jnp.broadcast_to(wgt[:, None], (e_pad, LANES))
        return s, d, wgt

    ww_s, ww_d, _ = pad_edges(edge_ww[0], edge_ww[1], None, 322560,
                              N_W, PW - N_W)
    wt_s, wt_d, wt_w = pad_edges(wt_src, wt_dst, w_wt, 161280,
                                 N_T, PT - N_T)
    wd_s, wd_d, wd_w = pad_edges(wd_src, wd_dst, w_wd, 322560,
                                 N_D, PD - N_D)
    td_s, td_d, td_w = pad_edges(td_src, td_dst, w_td, 64512,
                                 N_D, PD - N_D)

    word_h0 = _sc_gather(params['word_embeds'], word_id_p, PW, chunk=160)

    t0 = _tc_mm_scale(word_h0, params['gw_W0'], norm_p)
    p0 = _sc_agg_plain(t0, ww_s, ww_d, z_w, PW, chunk=112)

    t1 = _tc_post_mm(p0, norm_p, params['gw_b0'].reshape(1, D),
                     params['gw_W1'], out_scale=norm_p, out_bias=None)
    p1 = _sc_agg_plain(t1, ww_s, ww_d, z_w, PW, chunk=112)

    wcat = jnp.concatenate([params['h0_wt_W'], params['h1_wd_W']], axis=1)
    bcat = jnp.concatenate([params['h0_wt_b'], params['h1_wd_b']]).reshape(1, 2 * D)
    wh = _tc_post_mm(p1, norm_p, params['gw_b1'].reshape(1, D),
                     wcat, out_scale=None, out_bias=bcat)
    wh_wt = wh[:, :D]
    wh_wd = wh[:, D:]

    pt = _sc_agg_weighted(wh_wt, wt_s, wt_d, wt_w, z_t, msg_init,
                          PT, chunk=48)
    wh_td = _tc_mean_mm(pt.reshape(NC, PT, DC), params['h1_td_W'],
                        params['h1_td_b'].reshape(1, D))

    pwd = _sc_agg_weighted(wh_wd, wd_s, wd_d, wd_w, z_d, msg_init,
                           PD, chunk=48)
    wh_td_dep, _ = lax.optimization_barrier((wh_td, pwd))
    ptd = _sc_agg_weighted(wh_td_dep, td_s, td_d, td_w, z_d, msg_init,
                           PD, chunk=48)

    loss, y_pred = _tc_final(pwd.reshape(NC, PD, DC), ptd.reshape(NC, PD, DC),
                             ids_p,
                             params['out_W'].reshape(D, 1),
                             params['out_b'].reshape(1, 1),
                             y_data.reshape(B, 1))
    return loss.reshape(()), y_pred
